# baseline (speedup 1.0000x reference)
import jax
import jax.numpy as jnp
from jax.experimental import pallas as pl
import jax.experimental.pallas.tpu as pltpu

_B, _H, _KV, _Q, _DH = 8, 16, 2048, 16, 128
_BH = _B * _H
_NC = 8
_C = _BH // _NC


def _dma_kernel(pk_ref, pv_ref, ks_ref, vs_ref, ok_ref, ov_ref, sems):
    copies = []
    for i in range(_NC):
        rows = pl.ds(i * _C, _C)
        copies.append(pltpu.make_async_copy(
            pk_ref.at[rows], ok_ref.at[rows, pl.ds(0, _KV)], sems.at[2 * i]))
        copies.append(pltpu.make_async_copy(
            pv_ref.at[rows], ov_ref.at[rows, pl.ds(0, _KV)], sems.at[2 * i + 1]))
    copies.append(pltpu.make_async_copy(
        ks_ref, ok_ref.at[:, pl.ds(_KV, _Q)], sems.at[2 * _NC]))
    copies.append(pltpu.make_async_copy(
        vs_ref, ov_ref.at[:, pl.ds(_KV, _Q)], sems.at[2 * _NC + 1]))
    for c in copies:
        c.start()
    for c in copies:
        c.wait()


def kernel(past_key, past_value, key_states, value_states, layer_idx):
    pk = past_key.reshape(_BH, _KV, _DH)
    pv = past_value.reshape(_BH, _KV, _DH)
    ks = key_states.reshape(_BH, _Q, _DH)
    vs = value_states.reshape(_BH, _Q, _DH)

    hbm_spec = pl.BlockSpec(memory_space=pltpu.MemorySpace.HBM)
    out_shape = jax.ShapeDtypeStruct((_BH, _KV + _Q, _DH), jnp.float32)

    ok, ov = pl.pallas_call(
        _dma_kernel,
        in_specs=[hbm_spec] * 4,
        out_specs=[hbm_spec, hbm_spec],
        out_shape=[out_shape, out_shape],
        scratch_shapes=[pltpu.SemaphoreType.DMA((2 * _NC + 2,))],
    )(pk, pv, ks, vs)

    ok = ok.reshape(_B, _H, _KV + _Q, _DH)
    ov = ov.reshape(_B, _H, _KV + _Q, _DH)
    return (ok, ov)


# speedup vs baseline: 44.1400x; 44.1400x over previous
import jax
import jax.numpy as jnp
from jax.experimental import pallas as pl
import jax.experimental.pallas.tpu as pltpu

_B, _H, _KV, _Q, _DH = 8, 16, 2048, 16, 128
_BH = _B * _H
_R = 1


def _concat_copy(pk_ref, pv_ref, ks_ref, vs_ref, ok_ref, ov_ref):
    ok_ref[:, : _KV, :] = pk_ref[...]
    ok_ref[:, _KV :, :] = ks_ref[...]
    ov_ref[:, : _KV, :] = pv_ref[...]
    ov_ref[:, _KV :, :] = vs_ref[...]


def kernel(past_key, past_value, key_states, value_states, layer_idx):
    pk = past_key.reshape(_BH, _KV, _DH)
    pv = past_value.reshape(_BH, _KV, _DH)
    ks = key_states.reshape(_BH, _Q, _DH)
    vs = value_states.reshape(_BH, _Q, _DH)

    big_spec = pl.BlockSpec((_R, _KV, _DH), lambda i: (i, 0, 0))
    small_spec = pl.BlockSpec((_R, _Q, _DH), lambda i: (i, 0, 0))
    out_spec = pl.BlockSpec((_R, _KV + _Q, _DH), lambda i: (i, 0, 0))
    out_shape = jax.ShapeDtypeStruct((_BH, _KV + _Q, _DH), jnp.float32)

    ok, ov = pl.pallas_call(
        _concat_copy,
        grid=(_BH // _R,),
        in_specs=[big_spec, big_spec, small_spec, small_spec],
        out_specs=[out_spec, out_spec],
        out_shape=[out_shape, out_shape],
        compiler_params=pltpu.CompilerParams(
            dimension_semantics=("parallel",),
        ),
    )(pk, pv, ks, vs)

    ok = ok.reshape(_B, _H, _KV + _Q, _DH)
    ov = ov.reshape(_B, _H, _KV + _Q, _DH)
    return (ok, ov)


# speedup vs baseline: 49.0020x; 1.1101x over previous
import jax
import jax.numpy as jnp
from jax.experimental import pallas as pl
import jax.experimental.pallas.tpu as pltpu

_B, _H, _KV, _Q, _DH = 8, 16, 2048, 16, 128
_BH = _B * _H
_NBUF = 8
_L = 4


def _dma_pipeline(pk_ref, pv_ref, ks_ref, vs_ref, ok_ref, ov_ref,
                  kbuf, vbuf, kin, kout, vin, vout):
    def in_copies(i, s):
        return [
            pltpu.make_async_copy(pk_ref.at[i], kbuf.at[s, pl.ds(0, _KV)], kin.at[s]),
            pltpu.make_async_copy(ks_ref.at[i], kbuf.at[s, pl.ds(_KV, _Q)], kin.at[s]),
            pltpu.make_async_copy(pv_ref.at[i], vbuf.at[s, pl.ds(0, _KV)], vin.at[s]),
            pltpu.make_async_copy(vs_ref.at[i], vbuf.at[s, pl.ds(_KV, _Q)], vin.at[s]),
        ]

    def out_copies(i, s):
        return [
            pltpu.make_async_copy(kbuf.at[s], ok_ref.at[i], kout.at[s]),
            pltpu.make_async_copy(vbuf.at[s], ov_ref.at[i], vout.at[s]),
        ]

    for j in range(_L):
        for c in in_copies(j, j % _NBUF):
            c.start()
    for i in range(_BH):
        s = i % _NBUF
        nxt = i + _L
        if nxt < _BH:
            if nxt - _NBUF >= 0:
                for c in out_copies(nxt - _NBUF, nxt % _NBUF):
                    c.wait()
            for c in in_copies(nxt, nxt % _NBUF):
                c.start()
        for c in in_copies(i, s):
            c.wait()
        for c in out_copies(i, s):
            c.start()
    for j in range(_BH - _NBUF, _BH):
        for c in out_copies(j, j % _NBUF):
            c.wait()


def kernel(past_key, past_value, key_states, value_states, layer_idx):
    pk = past_key.reshape(_BH, _KV, _DH)
    pv = past_value.reshape(_BH, _KV, _DH)
    ks = key_states.reshape(_BH, _Q, _DH)
    vs = value_states.reshape(_BH, _Q, _DH)

    hbm_spec = pl.BlockSpec(memory_space=pltpu.MemorySpace.HBM)
    out_shape = jax.ShapeDtypeStruct((_BH, _KV + _Q, _DH), jnp.float32)

    ok, ov = pl.pallas_call(
        _dma_pipeline,
        in_specs=[hbm_spec] * 4,
        out_specs=[hbm_spec, hbm_spec],
        out_shape=[out_shape, out_shape],
        scratch_shapes=[
            pltpu.MemorySpace.VMEM((_NBUF, _KV + _Q, _DH), jnp.float32),
            pltpu.MemorySpace.VMEM((_NBUF, _KV + _Q, _DH), jnp.float32),
            pltpu.SemaphoreType.DMA((_NBUF,)),
            pltpu.SemaphoreType.DMA((_NBUF,)),
            pltpu.SemaphoreType.DMA((_NBUF,)),
            pltpu.SemaphoreType.DMA((_NBUF,)),
        ],
    )(pk, pv, ks, vs)

    ok = ok.reshape(_B, _H, _KV + _Q, _DH)
    ov = ov.reshape(_B, _H, _KV + _Q, _DH)
    return (ok, ov)
